# Initial kernel scaffold; baseline (speedup 1.0000x reference)
import jax
import jax.numpy as jnp
from jax.experimental import pallas as pl


def kernel(features0, features1, edge_index, edge_type, rel_x, rel_edge_index, rel_edge_attr, W1_0, b1_0, W1_1, b1_1, Wn, bn, W2, b2):
    raise NotImplementedError("write your pallas kernel here")


# speedup vs baseline: 4.2028x; 4.2028x over previous
import functools

import jax
import jax.numpy as jnp
from jax import lax
from jax.experimental import pallas as pl
from jax.experimental.pallas import tpu as pltpu
from jax.experimental.pallas import tpu_sc as plsc

NN = 10000
EE = 320000
DD = 128
RR = 8

NC = 2
NS = 16
NW = NC * NS
EPW = EE // NW
BB = 80
JBLK = 128
KK = 4
WB = 8
GW = JBLK // WB
NSUB = WB // KK
PAD_ROW = NN
RPT_A = 632
RPT_L = NN - (NS - 1) * RPT_A

BLK = 1000
NBLK = NN // BLK


def _wr_body(rx_ref, rd_ref, rs_ref, att_ref, wn_ref, bn_ref, wr_ref):
    oh_d = (lax.broadcasted_iota(jnp.int32, (RR, 64), 0) == rd_ref[:]).astype(jnp.float32)
    oh_s = (lax.broadcasted_iota(jnp.int32, (64, RR), 1) == rs_ref[:]).astype(jnp.float32)
    a = jnp.dot(oh_d * att_ref[:], oh_s, preferred_element_type=jnp.float32, precision=lax.Precision.HIGHEST)
    rx = rx_ref[:]
    rx = jnp.maximum(rx + jnp.dot(a, rx, preferred_element_type=jnp.float32, precision=lax.Precision.HIGHEST), 0.0)
    rx = jnp.maximum(rx + jnp.dot(a, rx, preferred_element_type=jnp.float32, precision=lax.Precision.HIGHEST), 0.0)
    wr_ref[:] = jnp.dot(rx, wn_ref[:], preferred_element_type=jnp.float32) + bn_ref[:]


def _compute_wr(rel_x, rel_edge_index, rel_edge_attr, wn, bn):
    rd_row = rel_edge_index[1:2, :].astype(jnp.int32)
    rs_col = rel_edge_index[0:1, :].astype(jnp.int32).reshape(64, 1)
    att_row = rel_edge_attr.reshape(1, 64)
    return pl.pallas_call(
        _wr_body,
        out_shape=jax.ShapeDtypeStruct((RR, DD), jnp.float32),
    )(rel_x, rd_row, rs_col, att_row, wn, bn.reshape(1, DD))


def _prologue_body(f_ref, w0_ref, b0_ref, w1_ref, b1_ref, wr_ref, h0_ref):
    i = pl.program_id(0)
    fb = f_ref[:]
    x0 = jnp.dot(fb, w0_ref[:], preferred_element_type=jnp.float32) + b0_ref[:]
    x1 = jnp.dot(fb, w1_ref[:], preferred_element_type=jnp.float32) + b1_ref[:]
    xb = jnp.where(i < (NBLK // 2), x0, x1)
    for r in range(RR):
        h0_ref[r] = wr_ref[r:r + 1, :] * xb


def _prologue(f, w1_0, b1_0, w1_1, b1_1, wr):
    return pl.pallas_call(
        _prologue_body,
        grid=(NBLK,),
        in_specs=[
            pl.BlockSpec((BLK, DD), lambda i: (i, 0)),
            pl.BlockSpec((DD, DD), lambda i: (0, 0)),
            pl.BlockSpec((1, DD), lambda i: (0, 0)),
            pl.BlockSpec((DD, DD), lambda i: (0, 0)),
            pl.BlockSpec((1, DD), lambda i: (0, 0)),
            pl.BlockSpec((RR, DD), lambda i: (0, 0)),
        ],
        out_specs=pl.BlockSpec((RR, BLK, DD), lambda i: (0, i, 0)),
        out_shape=jax.ShapeDtypeStruct((RR, NN, DD), jnp.float32),
    )(f, w1_0, b1_0.reshape(1, DD), w1_1, b1_1.reshape(1, DD), wr)


def _mid_body(p_ref, wr_ref, h1_ref):
    tb = jnp.tanh(p_ref[0] + p_ref[1])
    for r in range(RR):
        h1_ref[r] = wr_ref[r:r + 1, :] * tb


def _mid(p, wr):
    return pl.pallas_call(
        _mid_body,
        grid=(NBLK,),
        in_specs=[
            pl.BlockSpec((2, BLK, DD), lambda i: (0, i, 0)),
            pl.BlockSpec((RR, DD), lambda i: (0, 0)),
        ],
        out_specs=pl.BlockSpec((RR, BLK, DD), lambda i: (0, i, 0)),
        out_shape=jax.ShapeDtypeStruct((RR, NN, DD), jnp.float32),
    )(p, wr)


def _epilogue_body(p_ref, w2_ref, b2_ref, o_ref):
    tb = jnp.tanh(p_ref[0] + p_ref[1])
    o_ref[:] = jnp.dot(tb, w2_ref[:], preferred_element_type=jnp.float32) + b2_ref[:]


def _epilogue(p, w2, b2):
    c = w2.shape[1]
    return pl.pallas_call(
        _epilogue_body,
        grid=(NBLK,),
        in_specs=[
            pl.BlockSpec((2, BLK, DD), lambda i: (0, i, 0)),
            pl.BlockSpec((DD, c), lambda i: (0, 0)),
            pl.BlockSpec((1, c), lambda i: (0, 0)),
        ],
        out_specs=pl.BlockSpec((BLK, c), lambda i: (i, 0)),
        out_shape=jax.ShapeDtypeStruct((NN, c), jnp.float32),
    )(p, w2, b2.reshape(1, c))


def _sc_layer_body(h_hbm, gidx_hbm, didx_hbm, zeros_hbm, out_hbm,
                   gvw, dvw, ring, acc, sem, isem):
    c = lax.axis_index("c")
    s = lax.axis_index("s")
    wid = s * NC + c
    @pl.when(s < NS - 1)
    def _():
        pltpu.sync_copy(zeros_hbm.at[pl.ds(s * RPT_A, RPT_A)],
                        acc.at[pl.ds(s * RPT_A, RPT_A)])

    @pl.when(s == NS - 1)
    def _():
        pltpu.sync_copy(zeros_hbm.at[pl.ds((NS - 1) * RPT_A, RPT_L)],
                        acc.at[pl.ds((NS - 1) * RPT_A, RPT_L)])

    pltpu.async_copy(gidx_hbm.at[wid, pl.ds(0, WB)], gvw.at[pl.ds(0, WB)], isem)
    pltpu.async_copy(didx_hbm.at[wid, pl.ds(0, WB)], dvw.at[pl.ds(0, WB)], isem)
    plsc.subcore_barrier()

    def window(i, carry):
        h = (i % 2) * WB
        pltpu.make_async_copy(gidx_hbm.at[wid, pl.ds(0, WB)],
                              gvw.at[pl.ds(0, WB)], isem).wait()
        pltpu.make_async_copy(gidx_hbm.at[wid, pl.ds(0, WB)],
                              gvw.at[pl.ds(0, WB)], isem).wait()

        @pl.when(i + 1 < GW)
        def _():
            nh = ((i + 1) % 2) * WB
            pltpu.async_copy(gidx_hbm.at[wid, pl.ds((i + 1) * WB, WB)],
                             gvw.at[pl.ds(nh, WB)], isem)
            pltpu.async_copy(didx_hbm.at[wid, pl.ds((i + 1) * WB, WB)],
                             dvw.at[pl.ds(nh, WB)], isem)

        for sub in range(NSUB):
            for k in range(KK):
                pltpu.async_copy(h_hbm.at[gvw.at[h + sub * KK + k]],
                                 ring.at[k], sem)
            for _ in range(KK):
                pltpu.make_async_copy(h_hbm.at[gvw.at[0]], ring.at[0],
                                      sem).wait()
            for k in range(KK):
                pltpu.sync_copy(ring.at[k], acc.at[dvw.at[h + sub * KK + k]],
                                add=True)
        return carry

    lax.fori_loop(0, GW, window, 0)
    plsc.subcore_barrier()

    @pl.when(s < NS - 1)
    def _():
        pltpu.sync_copy(acc.at[pl.ds(s * RPT_A, RPT_A)],
                        out_hbm.at[pl.ds(c * NN + s * RPT_A, RPT_A)])

    @pl.when(s == NS - 1)
    def _():
        pltpu.sync_copy(acc.at[pl.ds((NS - 1) * RPT_A, RPT_L)],
                        out_hbm.at[pl.ds(c * NN + (NS - 1) * RPT_A, RPT_L)])


@functools.cache
def _make_sc_layer():
    return pl.kernel(
        _sc_layer_body,
        out_type=jax.ShapeDtypeStruct((NC * NN, DD), jnp.float32),
        mesh=plsc.VectorSubcoreMesh(core_axis_name="c", subcore_axis_name="s",
                                    num_cores=NC, num_subcores=NS),
        scratch_types=[
            pltpu.VMEM((2 * WB, BB), jnp.int32),
            pltpu.VMEM((2 * WB, BB), jnp.int32),
            pltpu.VMEM((KK, BB, DD), jnp.float32),
            pltpu.VMEM_SHARED((NN + 8, DD), jnp.float32),
            pltpu.SemaphoreType.DMA,
            pltpu.SemaphoreType.DMA,
        ],
    )


def _sc_layer(*args):
    return _make_sc_layer()(*args)


def kernel(features0, features1, edge_index, edge_type, rel_x, rel_edge_index,
           rel_edge_attr, W1_0, b1_0, W1_1, b1_1, Wn, bn, W2, b2):
    f = jnp.concatenate([features0, features1], axis=0)
    s_idx = edge_index[0].astype(jnp.int32)
    d_idx = edge_index[1].astype(jnp.int32)
    t_idx = edge_type.astype(jnp.int32)
    pad = JBLK * BB - EPW
    gidx = jnp.concatenate(
        [(t_idx * NN + s_idx).reshape(NW, EPW),
         jnp.zeros((NW, pad), jnp.int32)], axis=1).reshape(NW, JBLK, BB)
    didx = jnp.concatenate(
        [d_idx.reshape(NW, EPW),
         jnp.full((NW, pad), PAD_ROW, jnp.int32)], axis=1).reshape(NW, JBLK, BB)
    zeros = jnp.zeros((NN, DD), jnp.float32)

    wr = _compute_wr(rel_x, rel_edge_index, rel_edge_attr, Wn, bn)
    h0 = _prologue(f, W1_0, b1_0, W1_1, b1_1, wr).reshape(RR * NN, DD)
    p1 = _sc_layer(h0, gidx, didx, zeros).reshape(NC, NN, DD)
    h1 = _mid(p1, wr).reshape(RR * NN, DD)
    p2 = _sc_layer(h1, gidx, didx, zeros).reshape(NC, NN, DD)
    return _epilogue(p2, W2, b2)


# speedup vs baseline: 12.2778x; 2.9213x over previous
import functools

import jax
import jax.numpy as jnp
from jax import lax
from jax.experimental import pallas as pl
from jax.experimental.pallas import tpu as pltpu
from jax.experimental.pallas import tpu_sc as plsc

NN = 10000
EE = 320000
DD = 128
RR = 8

NC = 2
NS = 16
NW = NC * NS
EPW = EE // NW
BB = 80
JBLK = 128
KK = 4
WB = 8
GW = JBLK // WB
NSUB = WB // KK
PAD_ROW = NN
RPT_A = 632
RPT_L = NN - (NS - 1) * RPT_A

BLK = 1000
NBLK = NN // BLK


def _wr_body(rx_ref, rd_ref, rs_ref, att_ref, wn_ref, bn_ref, wr_ref):
    oh_d = (lax.broadcasted_iota(jnp.int32, (RR, 64), 0) == rd_ref[:]).astype(jnp.float32)
    oh_s = (lax.broadcasted_iota(jnp.int32, (64, RR), 1) == rs_ref[:]).astype(jnp.float32)
    a = jnp.dot(oh_d * att_ref[:], oh_s, preferred_element_type=jnp.float32, precision=lax.Precision.HIGHEST)
    rx = rx_ref[:]
    rx = jnp.maximum(rx + jnp.dot(a, rx, preferred_element_type=jnp.float32, precision=lax.Precision.HIGHEST), 0.0)
    rx = jnp.maximum(rx + jnp.dot(a, rx, preferred_element_type=jnp.float32, precision=lax.Precision.HIGHEST), 0.0)
    wr_ref[:] = jnp.dot(rx, wn_ref[:], preferred_element_type=jnp.float32) + bn_ref[:]


def _compute_wr(rel_x, rel_edge_index, rel_edge_attr, wn, bn):
    rd_row = rel_edge_index[1:2, :].astype(jnp.int32)
    rs_col = rel_edge_index[0:1, :].astype(jnp.int32).reshape(64, 1)
    att_row = rel_edge_attr.reshape(1, 64)
    return pl.pallas_call(
        _wr_body,
        out_shape=jax.ShapeDtypeStruct((RR, DD), jnp.float32),
    )(rel_x, rd_row, rs_col, att_row, wn, bn.reshape(1, DD))


def _prologue_body(f_ref, w0_ref, b0_ref, w1_ref, b1_ref, wr_ref, h0_ref):
    i = pl.program_id(0)
    fb = f_ref[:]
    x0 = jnp.dot(fb, w0_ref[:], preferred_element_type=jnp.float32) + b0_ref[:]
    x1 = jnp.dot(fb, w1_ref[:], preferred_element_type=jnp.float32) + b1_ref[:]
    xb = jnp.where(i < (NBLK // 2), x0, x1)
    for r in range(RR):
        h0_ref[r] = wr_ref[r:r + 1, :] * xb


def _prologue(f, w1_0, b1_0, w1_1, b1_1, wr):
    return pl.pallas_call(
        _prologue_body,
        grid=(NBLK,),
        in_specs=[
            pl.BlockSpec((BLK, DD), lambda i: (i, 0)),
            pl.BlockSpec((DD, DD), lambda i: (0, 0)),
            pl.BlockSpec((1, DD), lambda i: (0, 0)),
            pl.BlockSpec((DD, DD), lambda i: (0, 0)),
            pl.BlockSpec((1, DD), lambda i: (0, 0)),
            pl.BlockSpec((RR, DD), lambda i: (0, 0)),
        ],
        out_specs=pl.BlockSpec((RR, BLK, DD), lambda i: (0, i, 0)),
        out_shape=jax.ShapeDtypeStruct((RR, NN, DD), jnp.float32),
    )(f, w1_0, b1_0.reshape(1, DD), w1_1, b1_1.reshape(1, DD), wr)


def _mid_body(p_ref, wr_ref, h1_ref):
    tb = jnp.tanh(p_ref[0] + p_ref[1])
    for r in range(RR):
        h1_ref[r] = wr_ref[r:r + 1, :] * tb


def _mid(p, wr):
    return pl.pallas_call(
        _mid_body,
        grid=(NBLK,),
        in_specs=[
            pl.BlockSpec((2, BLK, DD), lambda i: (0, i, 0)),
            pl.BlockSpec((RR, DD), lambda i: (0, 0)),
        ],
        out_specs=pl.BlockSpec((RR, BLK, DD), lambda i: (0, i, 0)),
        out_shape=jax.ShapeDtypeStruct((RR, NN, DD), jnp.float32),
    )(p, wr)


def _epilogue_body(p_ref, w2_ref, b2_ref, o_ref):
    tb = jnp.tanh(p_ref[0] + p_ref[1])
    o_ref[:] = jnp.dot(tb, w2_ref[:], preferred_element_type=jnp.float32) + b2_ref[:]


def _epilogue(p, w2, b2):
    c = w2.shape[1]
    return pl.pallas_call(
        _epilogue_body,
        grid=(NBLK,),
        in_specs=[
            pl.BlockSpec((2, BLK, DD), lambda i: (0, i, 0)),
            pl.BlockSpec((DD, c), lambda i: (0, 0)),
            pl.BlockSpec((1, c), lambda i: (0, 0)),
        ],
        out_specs=pl.BlockSpec((BLK, c), lambda i: (i, 0)),
        out_shape=jax.ShapeDtypeStruct((NN, c), jnp.float32),
    )(p, w2, b2.reshape(1, c))


def _sc_layer_body(h_hbm, gidx_hbm, didx_hbm, zeros_hbm, out_hbm,
                   gvw, dvw, ring, acc,
                   gs0, gs1, gs2, gs3, ss0, ss1, ss2, ss3, isem):
    gs = [gs0, gs1, gs2, gs3]
    ss = [ss0, ss1, ss2, ss3]
    c = lax.axis_index("c")
    s = lax.axis_index("s")
    wid = s * NC + c
    @pl.when(s < NS - 1)
    def _():
        pltpu.sync_copy(zeros_hbm.at[pl.ds(s * RPT_A, RPT_A)],
                        acc.at[pl.ds(s * RPT_A, RPT_A)])

    @pl.when(s == NS - 1)
    def _():
        pltpu.sync_copy(zeros_hbm.at[pl.ds((NS - 1) * RPT_A, RPT_L)],
                        acc.at[pl.ds((NS - 1) * RPT_A, RPT_L)])

    def wait_idx():
        pltpu.make_async_copy(gidx_hbm.at[wid, pl.ds(0, WB)],
                              gvw.at[pl.ds(0, WB)], isem).wait()
        pltpu.make_async_copy(gidx_hbm.at[wid, pl.ds(0, WB)],
                              gvw.at[pl.ds(0, WB)], isem).wait()

    def fetch_idx(w):
        h = (w % 2) * WB
        pltpu.async_copy(gidx_hbm.at[wid, pl.ds(w * WB, WB)],
                         gvw.at[pl.ds(h, WB)], isem)
        pltpu.async_copy(didx_hbm.at[wid, pl.ds(w * WB, WB)],
                         dvw.at[pl.ds(h, WB)], isem)

    def fire_gather(idx_row, slot):
        pltpu.async_copy(h_hbm.at[gvw.at[idx_row]], ring.at[slot], gs[slot])

    def drain_gather(slot):
        pltpu.make_async_copy(h_hbm.at[gvw.at[0]], ring.at[0],
                              gs[slot]).wait()

    def fire_scatter(idx_row, slot):
        pltpu.async_copy(ring.at[slot], acc.at[dvw.at[idx_row]],
                         ss[slot], add=True)

    def wait_scatter(slot):
        pltpu.make_async_copy(ring.at[0], acc.at[dvw.at[0]],
                              ss[slot]).wait()

    fetch_idx(0)
    wait_idx()
    plsc.subcore_barrier()
    fire_gather(0, 0)
    fire_gather(1, 1)

    def round_(w, carry):
        h = (w % 2) * WB
        nh = ((w + 1) % 2) * WB

        for k in range(8):
            slot = k % 4
            drain_gather(slot)
            fire_scatter(h + k, slot)
            s2 = (k + 2) % 4
            if k < 2:
                @pl.when(w > 0)
                def _():
                    wait_scatter(s2)
                fire_gather(h + k + 2, s2)
            elif k < 6:
                if k == 2:
                    @pl.when(w < GW - 1)
                    def _():
                        fetch_idx(w + 1)
                wait_scatter(s2)
                fire_gather(h + k + 2, s2)
            else:
                wait_scatter(s2)
                if k == 6:
                    @pl.when(w < GW - 1)
                    def _():
                        wait_idx()
                        fire_gather(nh + 0, s2)
                else:
                    @pl.when(w < GW - 1)
                    def _():
                        fire_gather(nh + 1, s2)
        return carry

    lax.fori_loop(0, GW, round_, 0)
    wait_scatter(2)
    wait_scatter(3)
    plsc.subcore_barrier()

    @pl.when(s < NS - 1)
    def _():
        pltpu.sync_copy(acc.at[pl.ds(s * RPT_A, RPT_A)],
                        out_hbm.at[pl.ds(c * NN + s * RPT_A, RPT_A)])

    @pl.when(s == NS - 1)
    def _():
        pltpu.sync_copy(acc.at[pl.ds((NS - 1) * RPT_A, RPT_L)],
                        out_hbm.at[pl.ds(c * NN + (NS - 1) * RPT_A, RPT_L)])


@functools.cache
def _make_sc_layer():
    return pl.kernel(
        _sc_layer_body,
        out_type=jax.ShapeDtypeStruct((NC * NN, DD), jnp.float32),
        mesh=plsc.VectorSubcoreMesh(core_axis_name="c", subcore_axis_name="s",
                                    num_cores=NC, num_subcores=NS),
        scratch_types=[
            pltpu.VMEM((2 * WB, BB), jnp.int32),
            pltpu.VMEM((2 * WB, BB), jnp.int32),
            pltpu.VMEM((KK, BB, DD), jnp.float32),
            pltpu.VMEM_SHARED((NN + 8, DD), jnp.float32),
        ] + [pltpu.SemaphoreType.DMA] * 9,
    )


def _sc_layer(*args):
    return _make_sc_layer()(*args)


def kernel(features0, features1, edge_index, edge_type, rel_x, rel_edge_index,
           rel_edge_attr, W1_0, b1_0, W1_1, b1_1, Wn, bn, W2, b2):
    f = jnp.concatenate([features0, features1], axis=0)
    s_idx = edge_index[0].astype(jnp.int32)
    d_idx = edge_index[1].astype(jnp.int32)
    t_idx = edge_type.astype(jnp.int32)
    pad = JBLK * BB - EPW
    pad_g = (jnp.arange(NW * pad, dtype=jnp.int32) * 331) % (RR * NN)
    pad_d = PAD_ROW + (jnp.arange(NW * pad, dtype=jnp.int32) % 8)
    gidx = jnp.concatenate(
        [(t_idx * NN + s_idx).reshape(NW, EPW),
         pad_g.reshape(NW, pad)], axis=1).reshape(NW, JBLK, BB)
    didx = jnp.concatenate(
        [d_idx.reshape(NW, EPW),
         pad_d.reshape(NW, pad)], axis=1).reshape(NW, JBLK, BB)
    zeros = jnp.zeros((NN, DD), jnp.float32)

    wr = _compute_wr(rel_x, rel_edge_index, rel_edge_attr, Wn, bn)
    h0 = _prologue(f, W1_0, b1_0, W1_1, b1_1, wr).reshape(RR * NN, DD)
    p1 = _sc_layer(h0, gidx, didx, zeros).reshape(NC, NN, DD)
    h1 = _mid(p1, wr).reshape(RR * NN, DD)
    p2 = _sc_layer(h1, gidx, didx, zeros).reshape(NC, NN, DD)
    return _epilogue(p2, W2, b2)


# speedup vs baseline: 12.5223x; 1.0199x over previous
import functools

import jax
import jax.numpy as jnp
from jax import lax
from jax.experimental import pallas as pl
from jax.experimental.pallas import tpu as pltpu
from jax.experimental.pallas import tpu_sc as plsc

NN = 10000
EE = 320000
DD = 128
RR = 8

NC = 2
NS = 16
NW = NC * NS
EPW = EE // NW
BB = 80
JBLK = 128
KK = 4
WB = 8
GW = JBLK // WB
NSUB = WB // KK
PAD_ROW = NN
RPT_A = 632
RPT_L = NN - (NS - 1) * RPT_A

BLK = 1000
NBLK = NN // BLK


def _wr_body(rx_ref, rd_ref, rs_ref, att_ref, wn_ref, bn_ref, wr_ref):
    oh_d = (lax.broadcasted_iota(jnp.int32, (RR, 64), 0) == rd_ref[:]).astype(jnp.float32)
    oh_s = (lax.broadcasted_iota(jnp.int32, (64, RR), 1) == rs_ref[:]).astype(jnp.float32)
    a = jnp.dot(oh_d * att_ref[:], oh_s, preferred_element_type=jnp.float32, precision=lax.Precision.HIGHEST)
    rx = rx_ref[:]
    rx = jnp.maximum(rx + jnp.dot(a, rx, preferred_element_type=jnp.float32, precision=lax.Precision.HIGHEST), 0.0)
    rx = jnp.maximum(rx + jnp.dot(a, rx, preferred_element_type=jnp.float32, precision=lax.Precision.HIGHEST), 0.0)
    wr_ref[:] = jnp.dot(rx, wn_ref[:], preferred_element_type=jnp.float32) + bn_ref[:]


def _prologue_body(f0_ref, f1_ref, rx_ref, rd_ref, rs_ref, att_ref, wn_ref,
                   bn_ref, w0_ref, b0_ref, w1_ref, b1_ref,
                   h0_ref, wr_out_ref, wr_scr):
    i = pl.program_id(0)

    @pl.when(i == 0)
    def _():
        _wr_body(rx_ref, rd_ref, rs_ref, att_ref, wn_ref, bn_ref, wr_scr)
        wr_out_ref[:] = wr_scr[:]

    xb = jnp.where(
        i < (NBLK // 2),
        jnp.dot(f0_ref[:], w0_ref[:], preferred_element_type=jnp.float32) + b0_ref[:],
        jnp.dot(f1_ref[:], w1_ref[:], preferred_element_type=jnp.float32) + b1_ref[:])
    for r in range(RR):
        h0_ref[r] = wr_scr[r:r + 1, :] * xb


def _prologue(f0, f1, rel_x, rel_edge_index, rel_edge_attr, wn, bn,
              w1_0, b1_0, w1_1, b1_1):
    rd_row = rel_edge_index[1:2, :].astype(jnp.int32)
    rs_col = rel_edge_index[0:1, :].astype(jnp.int32).reshape(64, 1)
    att_row = rel_edge_attr.reshape(1, 64)
    half = NBLK // 2
    return pl.pallas_call(
        _prologue_body,
        grid=(NBLK,),
        in_specs=[
            pl.BlockSpec((BLK, DD), lambda i: (jnp.minimum(i, half - 1), 0)),
            pl.BlockSpec((BLK, DD),
                         lambda i: (jnp.maximum(i - half, 0), 0)),
            pl.BlockSpec((RR, RR), lambda i: (0, 0)),
            pl.BlockSpec((1, 64), lambda i: (0, 0)),
            pl.BlockSpec((64, 1), lambda i: (0, 0)),
            pl.BlockSpec((1, 64), lambda i: (0, 0)),
            pl.BlockSpec((RR, DD), lambda i: (0, 0)),
            pl.BlockSpec((1, DD), lambda i: (0, 0)),
            pl.BlockSpec((DD, DD), lambda i: (0, 0)),
            pl.BlockSpec((1, DD), lambda i: (0, 0)),
            pl.BlockSpec((DD, DD), lambda i: (0, 0)),
            pl.BlockSpec((1, DD), lambda i: (0, 0)),
        ],
        out_specs=[
            pl.BlockSpec((RR, BLK, DD), lambda i: (0, i, 0)),
            pl.BlockSpec((RR, DD), lambda i: (0, 0)),
        ],
        out_shape=[
            jax.ShapeDtypeStruct((RR, NN, DD), jnp.float32),
            jax.ShapeDtypeStruct((RR, DD), jnp.float32),
        ],
        scratch_shapes=[pltpu.VMEM((RR, DD), jnp.float32)],
    )(f0, f1, rel_x, rd_row, rs_col, att_row, wn, bn.reshape(1, DD),
      w1_0, b1_0.reshape(1, DD), w1_1, b1_1.reshape(1, DD))


def _mid_body(p_ref, wr_ref, h1_ref):
    tb = jnp.tanh(p_ref[0] + p_ref[1])
    for r in range(RR):
        h1_ref[r] = wr_ref[r:r + 1, :] * tb


def _mid(p, wr):
    return pl.pallas_call(
        _mid_body,
        grid=(NBLK,),
        in_specs=[
            pl.BlockSpec((2, BLK, DD), lambda i: (0, i, 0)),
            pl.BlockSpec((RR, DD), lambda i: (0, 0)),
        ],
        out_specs=pl.BlockSpec((RR, BLK, DD), lambda i: (0, i, 0)),
        out_shape=jax.ShapeDtypeStruct((RR, NN, DD), jnp.float32),
    )(p, wr)


def _epilogue_body(p_ref, w2_ref, b2_ref, o_ref):
    tb = jnp.tanh(p_ref[0] + p_ref[1])
    o_ref[:] = jnp.dot(tb, w2_ref[:], preferred_element_type=jnp.float32) + b2_ref[:]


def _epilogue(p, w2, b2):
    c = w2.shape[1]
    return pl.pallas_call(
        _epilogue_body,
        grid=(NBLK,),
        in_specs=[
            pl.BlockSpec((2, BLK, DD), lambda i: (0, i, 0)),
            pl.BlockSpec((DD, c), lambda i: (0, 0)),
            pl.BlockSpec((1, c), lambda i: (0, 0)),
        ],
        out_specs=pl.BlockSpec((BLK, c), lambda i: (i, 0)),
        out_shape=jax.ShapeDtypeStruct((NN, c), jnp.float32),
    )(p, w2, b2.reshape(1, c))


def _sc_layer_body(h_hbm, gidx_hbm, didx_hbm, zeros_hbm, out_hbm,
                   gvw, dvw, ring, acc,
                   gs0, gs1, gs2, gs3, ss0, ss1, ss2, ss3, isem):
    gs = [gs0, gs1, gs2, gs3]
    ss = [ss0, ss1, ss2, ss3]
    c = lax.axis_index("c")
    s = lax.axis_index("s")
    wid = s * NC + c
    @pl.when(s < NS - 1)
    def _():
        pltpu.sync_copy(zeros_hbm.at[pl.ds(s * RPT_A, RPT_A)],
                        acc.at[pl.ds(s * RPT_A, RPT_A)])

    @pl.when(s == NS - 1)
    def _():
        pltpu.sync_copy(zeros_hbm.at[pl.ds((NS - 1) * RPT_A, RPT_L)],
                        acc.at[pl.ds((NS - 1) * RPT_A, RPT_L)])

    def wait_idx():
        pltpu.make_async_copy(gidx_hbm.at[wid, pl.ds(0, WB)],
                              gvw.at[pl.ds(0, WB)], isem).wait()
        pltpu.make_async_copy(gidx_hbm.at[wid, pl.ds(0, WB)],
                              gvw.at[pl.ds(0, WB)], isem).wait()

    def fetch_idx(w):
        h = (w % 2) * WB
        pltpu.async_copy(gidx_hbm.at[wid, pl.ds(w * WB, WB)],
                         gvw.at[pl.ds(h, WB)], isem)
        pltpu.async_copy(didx_hbm.at[wid, pl.ds(w * WB, WB)],
                         dvw.at[pl.ds(h, WB)], isem)

    def fire_gather(idx_row, slot):
        pltpu.async_copy(h_hbm.at[gvw.at[idx_row]], ring.at[slot], gs[slot])

    def drain_gather(slot):
        pltpu.make_async_copy(h_hbm.at[gvw.at[0]], ring.at[0],
                              gs[slot]).wait()

    def fire_scatter(idx_row, slot):
        pltpu.async_copy(ring.at[slot], acc.at[dvw.at[idx_row]],
                         ss[slot], add=True)

    def wait_scatter(slot):
        pltpu.make_async_copy(ring.at[0], acc.at[dvw.at[0]],
                              ss[slot]).wait()

    fetch_idx(0)
    wait_idx()
    plsc.subcore_barrier()
    fire_gather(0, 0)
    fire_gather(1, 1)

    def round_(w, carry):
        h = (w % 2) * WB
        nh = ((w + 1) % 2) * WB

        for k in range(8):
            slot = k % 4
            drain_gather(slot)
            fire_scatter(h + k, slot)
            s2 = (k + 2) % 4
            if k < 2:
                @pl.when(w > 0)
                def _():
                    wait_scatter(s2)
                fire_gather(h + k + 2, s2)
            elif k < 6:
                if k == 2:
                    @pl.when(w < GW - 1)
                    def _():
                        fetch_idx(w + 1)
                wait_scatter(s2)
                fire_gather(h + k + 2, s2)
            else:
                wait_scatter(s2)
                if k == 6:
                    @pl.when(w < GW - 1)
                    def _():
                        wait_idx()
                        fire_gather(nh + 0, s2)
                else:
                    @pl.when(w < GW - 1)
                    def _():
                        fire_gather(nh + 1, s2)
        return carry

    lax.fori_loop(0, GW, round_, 0)
    wait_scatter(2)
    wait_scatter(3)
    plsc.subcore_barrier()

    @pl.when(s < NS - 1)
    def _():
        pltpu.sync_copy(acc.at[pl.ds(s * RPT_A, RPT_A)],
                        out_hbm.at[pl.ds(c * NN + s * RPT_A, RPT_A)])

    @pl.when(s == NS - 1)
    def _():
        pltpu.sync_copy(acc.at[pl.ds((NS - 1) * RPT_A, RPT_L)],
                        out_hbm.at[pl.ds(c * NN + (NS - 1) * RPT_A, RPT_L)])


@functools.cache
def _make_sc_layer():
    return pl.kernel(
        _sc_layer_body,
        out_type=jax.ShapeDtypeStruct((NC * NN, DD), jnp.float32),
        mesh=plsc.VectorSubcoreMesh(core_axis_name="c", subcore_axis_name="s",
                                    num_cores=NC, num_subcores=NS),
        scratch_types=[
            pltpu.VMEM((2 * WB, BB), jnp.int32),
            pltpu.VMEM((2 * WB, BB), jnp.int32),
            pltpu.VMEM((KK, BB, DD), jnp.float32),
            pltpu.VMEM_SHARED((NN + 8, DD), jnp.float32),
        ] + [pltpu.SemaphoreType.DMA] * 9,
    )


def _sc_layer(*args):
    return _make_sc_layer()(*args)


def kernel(features0, features1, edge_index, edge_type, rel_x, rel_edge_index,
           rel_edge_attr, W1_0, b1_0, W1_1, b1_1, Wn, bn, W2, b2):
    s_idx = edge_index[0].astype(jnp.int32)
    d_idx = edge_index[1].astype(jnp.int32)
    t_idx = edge_type.astype(jnp.int32)
    pad = JBLK * BB - EPW
    pad_g = (jnp.arange(NW * pad, dtype=jnp.int32) * 331) % (RR * NN)
    pad_d = PAD_ROW + (jnp.arange(NW * pad, dtype=jnp.int32) % 8)
    gidx = jnp.concatenate(
        [(t_idx * NN + s_idx).reshape(NW, EPW),
         pad_g.reshape(NW, pad)], axis=1).reshape(NW, JBLK, BB)
    didx = jnp.concatenate(
        [d_idx.reshape(NW, EPW),
         pad_d.reshape(NW, pad)], axis=1).reshape(NW, JBLK, BB)
    zeros = jnp.zeros((NN, DD), jnp.float32)

    h0, wr = _prologue(features0, features1, rel_x, rel_edge_index,
                       rel_edge_attr, Wn, bn, W1_0, b1_0, W1_1, b1_1)
    h0 = h0.reshape(RR * NN, DD)
    p1 = _sc_layer(h0, gidx, didx, zeros).reshape(NC, NN, DD)
    h1 = _mid(p1, wr).reshape(RR * NN, DD)
    p2 = _sc_layer(h1, gidx, didx, zeros).reshape(NC, NN, DD)
    return _epilogue(p2, W2, b2)


# speedup vs baseline: 12.5331x; 1.0009x over previous
import functools

import jax
import jax.numpy as jnp
from jax import lax
from jax.experimental import pallas as pl
from jax.experimental.pallas import tpu as pltpu
from jax.experimental.pallas import tpu_sc as plsc

NN = 10000
EE = 320000
DD = 128
RR = 8

NC = 2
NS = 16
NW = NC * NS
EPW = EE // NW
BB = 40
JBLK = 256
KK = 8
WB = 8
GW = JBLK // WB
PAD_ROW = NN
RPT_A = 632
RPT_L = NN - (NS - 1) * RPT_A

BLK = 1000
NBLK = NN // BLK


def _wr_body(rx_ref, rd_ref, rs_ref, att_ref, wn_ref, bn_ref, wr_ref):
    oh_d = (lax.broadcasted_iota(jnp.int32, (RR, 64), 0) == rd_ref[:]).astype(jnp.float32)
    oh_s = (lax.broadcasted_iota(jnp.int32, (64, RR), 1) == rs_ref[:]).astype(jnp.float32)
    a = jnp.dot(oh_d * att_ref[:], oh_s, preferred_element_type=jnp.float32, precision=lax.Precision.HIGHEST)
    rx = rx_ref[:]
    rx = jnp.maximum(rx + jnp.dot(a, rx, preferred_element_type=jnp.float32, precision=lax.Precision.HIGHEST), 0.0)
    rx = jnp.maximum(rx + jnp.dot(a, rx, preferred_element_type=jnp.float32, precision=lax.Precision.HIGHEST), 0.0)
    wr_ref[:] = jnp.dot(rx, wn_ref[:], preferred_element_type=jnp.float32) + bn_ref[:]


def _compute_wr(rel_x, rel_edge_index, rel_edge_attr, wn, bn):
    rd_row = rel_edge_index[1:2, :].astype(jnp.int32)
    rs_col = rel_edge_index[0:1, :].astype(jnp.int32).reshape(64, 1)
    att_row = rel_edge_attr.reshape(1, 64)
    return pl.pallas_call(
        _wr_body,
        out_shape=jax.ShapeDtypeStruct((RR, DD), jnp.float32),
    )(rel_x, rd_row, rs_col, att_row, wn, bn.reshape(1, DD))


def _prologue_body(f_ref, w0_ref, b0_ref, w1_ref, b1_ref, wr_ref, h0_ref):
    i = pl.program_id(0)
    fb = f_ref[:]
    x0 = jnp.dot(fb, w0_ref[:], preferred_element_type=jnp.float32) + b0_ref[:]
    x1 = jnp.dot(fb, w1_ref[:], preferred_element_type=jnp.float32) + b1_ref[:]
    xb = jnp.where(i < (NBLK // 2), x0, x1)
    for r in range(RR):
        h0_ref[r] = wr_ref[r:r + 1, :] * xb


def _prologue(f, w1_0, b1_0, w1_1, b1_1, wr):
    return pl.pallas_call(
        _prologue_body,
        grid=(NBLK,),
        in_specs=[
            pl.BlockSpec((BLK, DD), lambda i: (i, 0)),
            pl.BlockSpec((DD, DD), lambda i: (0, 0)),
            pl.BlockSpec((1, DD), lambda i: (0, 0)),
            pl.BlockSpec((DD, DD), lambda i: (0, 0)),
            pl.BlockSpec((1, DD), lambda i: (0, 0)),
            pl.BlockSpec((RR, DD), lambda i: (0, 0)),
        ],
        out_specs=pl.BlockSpec((RR, BLK, DD), lambda i: (0, i, 0)),
        out_shape=jax.ShapeDtypeStruct((RR, NN, DD), jnp.float32),
    )(f, w1_0, b1_0.reshape(1, DD), w1_1, b1_1.reshape(1, DD), wr)


def _mid_body(p_ref, wr_ref, h1_ref):
    tb = jnp.tanh(p_ref[0] + p_ref[1])
    for r in range(RR):
        h1_ref[r] = wr_ref[r:r + 1, :] * tb


def _mid(p, wr):
    return pl.pallas_call(
        _mid_body,
        grid=(NBLK,),
        in_specs=[
            pl.BlockSpec((2, BLK, DD), lambda i: (0, i, 0)),
            pl.BlockSpec((RR, DD), lambda i: (0, 0)),
        ],
        out_specs=pl.BlockSpec((RR, BLK, DD), lambda i: (0, i, 0)),
        out_shape=jax.ShapeDtypeStruct((RR, NN, DD), jnp.float32),
    )(p, wr)


def _epilogue_body(p_ref, w2_ref, b2_ref, o_ref):
    tb = jnp.tanh(p_ref[0] + p_ref[1])
    o_ref[:] = jnp.dot(tb, w2_ref[:], preferred_element_type=jnp.float32) + b2_ref[:]


def _epilogue(p, w2, b2):
    c = w2.shape[1]
    return pl.pallas_call(
        _epilogue_body,
        grid=(NBLK,),
        in_specs=[
            pl.BlockSpec((2, BLK, DD), lambda i: (0, i, 0)),
            pl.BlockSpec((DD, c), lambda i: (0, 0)),
            pl.BlockSpec((1, c), lambda i: (0, 0)),
        ],
        out_specs=pl.BlockSpec((BLK, c), lambda i: (i, 0)),
        out_shape=jax.ShapeDtypeStruct((NN, c), jnp.float32),
    )(p, w2, b2.reshape(1, c))


def _sc_layer_body(h_hbm, gidx_hbm, didx_hbm, zeros_hbm, out_hbm,
                   gvw, dvw, ring, acc,
                   gs0, gs1, gs2, gs3, gs4, gs5, gs6, gs7,
                   ss0, ss1, ss2, ss3, ss4, ss5, ss6, ss7, isg, isd):
    gs = [gs0, gs1, gs2, gs3, gs4, gs5, gs6, gs7]
    ss = [ss0, ss1, ss2, ss3, ss4, ss5, ss6, ss7]
    c = lax.axis_index("c")
    s = lax.axis_index("s")
    wid = s * NC + c
    @pl.when(s < NS - 1)
    def _():
        pltpu.sync_copy(zeros_hbm.at[pl.ds(s * RPT_A, RPT_A)],
                        acc.at[pl.ds(s * RPT_A, RPT_A)])

    @pl.when(s == NS - 1)
    def _():
        pltpu.sync_copy(zeros_hbm.at[pl.ds((NS - 1) * RPT_A, RPT_L)],
                        acc.at[pl.ds((NS - 1) * RPT_A, RPT_L)])

    def wait_idx_g():
        pltpu.make_async_copy(gidx_hbm.at[wid, pl.ds(0, WB)],
                              gvw.at[pl.ds(0, WB)], isg).wait()

    def wait_idx_d():
        pltpu.make_async_copy(didx_hbm.at[wid, pl.ds(0, WB)],
                              dvw.at[pl.ds(0, WB)], isd).wait()

    def fetch_idx_g(w):
        pltpu.async_copy(gidx_hbm.at[wid, pl.ds(w * WB, WB)],
                         gvw.at[pl.ds((w % 2) * WB, WB)], isg)

    def fetch_idx_d(w):
        pltpu.async_copy(didx_hbm.at[wid, pl.ds(w * WB, WB)],
                         dvw.at[pl.ds((w % 2) * WB, WB)], isd)

    def fire_gather(idx_row, slot):
        pltpu.async_copy(h_hbm.at[gvw.at[idx_row]], ring.at[slot], gs[slot])

    def drain_gather(slot):
        pltpu.make_async_copy(h_hbm.at[gvw.at[0]], ring.at[0],
                              gs[slot]).wait()

    def fire_scatter(idx_row, slot):
        pltpu.async_copy(ring.at[slot], acc.at[dvw.at[idx_row]],
                         ss[slot], add=True)

    def wait_scatter(slot):
        pltpu.make_async_copy(ring.at[0], acc.at[dvw.at[0]],
                              ss[slot]).wait()

    fetch_idx_g(0)
    fetch_idx_d(0)
    wait_idx_g()
    wait_idx_d()
    plsc.subcore_barrier()
    for k in range(4):
        fire_gather(k, k)

    def round_(w, carry):
        h = (w % 2) * WB
        nh = ((w + 1) % 2) * WB

        for k in range(8):
            drain_gather(k)
            fire_scatter(h + k, k)
            s2 = (k + 4) % 8
            if k == 0:
                @pl.when(w < GW - 1)
                def _():
                    fetch_idx_g(w + 1)
            if k < 4:
                @pl.when(w > 0)
                def _():
                    wait_scatter(s2)
                fire_gather(h + k + 4, s2)
            else:
                wait_scatter(s2)
                if k == 4:
                    @pl.when(w < GW - 1)
                    def _():
                        fetch_idx_d(w + 1)
                        wait_idx_g()
                @pl.when(w < GW - 1)
                def _():
                    fire_gather(nh + k - 4, s2)
                if k == 7:
                    @pl.when(w < GW - 1)
                    def _():
                        wait_idx_d()
        return carry

    lax.fori_loop(0, GW, round_, 0)
    for k in range(4, 8):
        wait_scatter(k)
    plsc.subcore_barrier()

    @pl.when(s < NS - 1)
    def _():
        pltpu.sync_copy(acc.at[pl.ds(s * RPT_A, RPT_A)],
                        out_hbm.at[pl.ds(c * NN + s * RPT_A, RPT_A)])

    @pl.when(s == NS - 1)
    def _():
        pltpu.sync_copy(acc.at[pl.ds((NS - 1) * RPT_A, RPT_L)],
                        out_hbm.at[pl.ds(c * NN + (NS - 1) * RPT_A, RPT_L)])


@functools.cache
def _make_sc_layer():
    return pl.kernel(
        _sc_layer_body,
        out_type=jax.ShapeDtypeStruct((NC * NN, DD), jnp.float32),
        mesh=plsc.VectorSubcoreMesh(core_axis_name="c", subcore_axis_name="s",
                                    num_cores=NC, num_subcores=NS),
        scratch_types=[
            pltpu.VMEM((2 * WB, BB), jnp.int32),
            pltpu.VMEM((2 * WB, BB), jnp.int32),
            pltpu.VMEM((KK, BB, DD), jnp.float32),
            pltpu.VMEM_SHARED((NN + 8, DD), jnp.float32),
        ] + [pltpu.SemaphoreType.DMA] * 18,
    )


def _sc_layer(*args):
    return _make_sc_layer()(*args)


def kernel(features0, features1, edge_index, edge_type, rel_x, rel_edge_index,
           rel_edge_attr, W1_0, b1_0, W1_1, b1_1, Wn, bn, W2, b2):
    f = jnp.concatenate([features0, features1], axis=0)
    s_idx = edge_index[0].astype(jnp.int32)
    d_idx = edge_index[1].astype(jnp.int32)
    t_idx = edge_type.astype(jnp.int32)
    pad = JBLK * BB - EPW
    pad_g = (jnp.arange(NW * pad, dtype=jnp.int32) * 331) % (RR * NN)
    pad_d = PAD_ROW + (jnp.arange(NW * pad, dtype=jnp.int32) % 8)
    gidx = jnp.concatenate(
        [(t_idx * NN + s_idx).reshape(NW, EPW),
         pad_g.reshape(NW, pad)], axis=1).reshape(NW, JBLK, BB)
    didx = jnp.concatenate(
        [d_idx.reshape(NW, EPW),
         pad_d.reshape(NW, pad)], axis=1).reshape(NW, JBLK, BB)
    zeros = jnp.zeros((NN, DD), jnp.float32)

    wr = _compute_wr(rel_x, rel_edge_index, rel_edge_attr, Wn, bn)
    h0 = _prologue(f, W1_0, b1_0, W1_1, b1_1, wr).reshape(RR * NN, DD)
    p1 = _sc_layer(h0, gidx, didx, zeros).reshape(NC, NN, DD)
    h1 = _mid(p1, wr).reshape(RR * NN, DD)
    p2 = _sc_layer(h1, gidx, didx, zeros).reshape(NC, NN, DD)
    return _epilogue(p2, W2, b2)


# speedup vs baseline: 12.7841x; 1.0200x over previous
import functools

import jax
import jax.numpy as jnp
from jax import lax
from jax.experimental import pallas as pl
from jax.experimental.pallas import tpu as pltpu
from jax.experimental.pallas import tpu_sc as plsc

NN = 10000
EE = 320000
DD = 128
RR = 8

NC = 2
NS = 16
NW = NC * NS
EPW = EE // NW
BB = 40
JBLK = 256
KK = 8
WB = 8
GW = JBLK // WB
PAD_ROW = NN
RPT_A = 632
RPT_L = NN - (NS - 1) * RPT_A

BLK = 1000
NBLK = NN // BLK


def _wr_body(rx_ref, rd_ref, rs_ref, att_ref, wn_ref, bn_ref, wr_ref):
    oh_d = (lax.broadcasted_iota(jnp.int32, (RR, 64), 0) == rd_ref[:]).astype(jnp.float32)
    oh_s = (lax.broadcasted_iota(jnp.int32, (64, RR), 1) == rs_ref[:]).astype(jnp.float32)
    a = jnp.dot(oh_d * att_ref[:], oh_s, preferred_element_type=jnp.float32, precision=lax.Precision.HIGHEST)
    rx = rx_ref[:]
    rx = jnp.maximum(rx + jnp.dot(a, rx, preferred_element_type=jnp.float32, precision=lax.Precision.HIGHEST), 0.0)
    rx = jnp.maximum(rx + jnp.dot(a, rx, preferred_element_type=jnp.float32, precision=lax.Precision.HIGHEST), 0.0)
    wr_ref[:] = jnp.dot(rx, wn_ref[:], preferred_element_type=jnp.float32) + bn_ref[:]


def _prologue_body(f0_ref, f1_ref, rx_ref, rd_ref, rs_ref, att_ref, wn_ref,
                   bn_ref, w0_ref, b0_ref, w1_ref, b1_ref,
                   h0_ref, wr_out_ref, wr_scr):
    i = pl.program_id(0)

    @pl.when(i == 0)
    def _():
        _wr_body(rx_ref, rd_ref, rs_ref, att_ref, wn_ref, bn_ref, wr_scr)
        wr_out_ref[:] = wr_scr[:]

    xb = jnp.where(
        i < (NBLK // 2),
        jnp.dot(f0_ref[:], w0_ref[:], preferred_element_type=jnp.float32) + b0_ref[:],
        jnp.dot(f1_ref[:], w1_ref[:], preferred_element_type=jnp.float32) + b1_ref[:])
    for r in range(RR):
        h0_ref[r] = wr_scr[r:r + 1, :] * xb


def _prologue(f0, f1, rel_x, rel_edge_index, rel_edge_attr, wn, bn,
              w1_0, b1_0, w1_1, b1_1):
    rd_row = rel_edge_index[1:2, :].astype(jnp.int32)
    rs_col = rel_edge_index[0:1, :].astype(jnp.int32).reshape(64, 1)
    att_row = rel_edge_attr.reshape(1, 64)
    half = NBLK // 2
    return pl.pallas_call(
        _prologue_body,
        grid=(NBLK,),
        in_specs=[
            pl.BlockSpec((BLK, DD), lambda i: (jnp.minimum(i, half - 1), 0)),
            pl.BlockSpec((BLK, DD),
                         lambda i: (jnp.maximum(i - half, 0), 0)),
            pl.BlockSpec((RR, RR), lambda i: (0, 0)),
            pl.BlockSpec((1, 64), lambda i: (0, 0)),
            pl.BlockSpec((64, 1), lambda i: (0, 0)),
            pl.BlockSpec((1, 64), lambda i: (0, 0)),
            pl.BlockSpec((RR, DD), lambda i: (0, 0)),
            pl.BlockSpec((1, DD), lambda i: (0, 0)),
            pl.BlockSpec((DD, DD), lambda i: (0, 0)),
            pl.BlockSpec((1, DD), lambda i: (0, 0)),
            pl.BlockSpec((DD, DD), lambda i: (0, 0)),
            pl.BlockSpec((1, DD), lambda i: (0, 0)),
        ],
        out_specs=[
            pl.BlockSpec((RR, BLK, DD), lambda i: (0, i, 0)),
            pl.BlockSpec((RR, DD), lambda i: (0, 0)),
        ],
        out_shape=[
            jax.ShapeDtypeStruct((RR, NN, DD), jnp.float32),
            jax.ShapeDtypeStruct((RR, DD), jnp.float32),
        ],
        scratch_shapes=[pltpu.VMEM((RR, DD), jnp.float32)],
    )(f0, f1, rel_x, rd_row, rs_col, att_row, wn, bn.reshape(1, DD),
      w1_0, b1_0.reshape(1, DD), w1_1, b1_1.reshape(1, DD))


def _mid_body(p_ref, wr_ref, h1_ref):
    tb = jnp.tanh(p_ref[0] + p_ref[1])
    for r in range(RR):
        h1_ref[r] = wr_ref[r:r + 1, :] * tb


def _mid(p, wr):
    return pl.pallas_call(
        _mid_body,
        grid=(NBLK,),
        in_specs=[
            pl.BlockSpec((2, BLK, DD), lambda i: (0, i, 0)),
            pl.BlockSpec((RR, DD), lambda i: (0, 0)),
        ],
        out_specs=pl.BlockSpec((RR, BLK, DD), lambda i: (0, i, 0)),
        out_shape=jax.ShapeDtypeStruct((RR, NN, DD), jnp.float32),
    )(p, wr)


def _epilogue_body(p_ref, w2_ref, b2_ref, o_ref):
    tb = jnp.tanh(p_ref[0] + p_ref[1])
    o_ref[:] = jnp.dot(tb, w2_ref[:], preferred_element_type=jnp.float32) + b2_ref[:]


def _epilogue(p, w2, b2):
    c = w2.shape[1]
    return pl.pallas_call(
        _epilogue_body,
        grid=(NBLK,),
        in_specs=[
            pl.BlockSpec((2, BLK, DD), lambda i: (0, i, 0)),
            pl.BlockSpec((DD, c), lambda i: (0, 0)),
            pl.BlockSpec((1, c), lambda i: (0, 0)),
        ],
        out_specs=pl.BlockSpec((BLK, c), lambda i: (i, 0)),
        out_shape=jax.ShapeDtypeStruct((NN, c), jnp.float32),
    )(p, w2, b2.reshape(1, c))


def _sc_layer_body(h_hbm, gidx_hbm, didx_hbm, zeros_hbm, out_hbm,
                   gvw, dvw, ring, acc,
                   gs0, gs1, gs2, gs3, gs4, gs5, gs6, gs7,
                   ss0, ss1, ss2, ss3, ss4, ss5, ss6, ss7, isg, isd):
    gs = [gs0, gs1, gs2, gs3, gs4, gs5, gs6, gs7]
    ss = [ss0, ss1, ss2, ss3, ss4, ss5, ss6, ss7]
    c = lax.axis_index("c")
    s = lax.axis_index("s")
    wid = s * NC + c
    @pl.when(s < NS - 1)
    def _():
        pltpu.sync_copy(zeros_hbm.at[pl.ds(s * RPT_A, RPT_A)],
                        acc.at[pl.ds(s * RPT_A, RPT_A)])

    @pl.when(s == NS - 1)
    def _():
        pltpu.sync_copy(zeros_hbm.at[pl.ds((NS - 1) * RPT_A, RPT_L)],
                        acc.at[pl.ds((NS - 1) * RPT_A, RPT_L)])

    def wait_idx_g():
        pltpu.make_async_copy(gidx_hbm.at[wid, pl.ds(0, WB)],
                              gvw.at[pl.ds(0, WB)], isg).wait()

    def wait_idx_d():
        pltpu.make_async_copy(didx_hbm.at[wid, pl.ds(0, WB)],
                              dvw.at[pl.ds(0, WB)], isd).wait()

    def fetch_idx_g(w):
        pltpu.async_copy(gidx_hbm.at[wid, pl.ds(w * WB, WB)],
                         gvw.at[pl.ds((w % 2) * WB, WB)], isg)

    def fetch_idx_d(w):
        pltpu.async_copy(didx_hbm.at[wid, pl.ds(w * WB, WB)],
                         dvw.at[pl.ds((w % 2) * WB, WB)], isd)

    def fire_gather(idx_row, slot):
        pltpu.async_copy(h_hbm.at[gvw.at[idx_row]], ring.at[slot], gs[slot])

    def drain_gather(slot):
        pltpu.make_async_copy(h_hbm.at[gvw.at[0]], ring.at[0],
                              gs[slot]).wait()

    def fire_scatter(idx_row, slot):
        pltpu.async_copy(ring.at[slot], acc.at[dvw.at[idx_row]],
                         ss[slot], add=True)

    def wait_scatter(slot):
        pltpu.make_async_copy(ring.at[0], acc.at[dvw.at[0]],
                              ss[slot]).wait()

    fetch_idx_g(0)
    fetch_idx_d(0)
    wait_idx_g()
    wait_idx_d()
    plsc.subcore_barrier()
    for k in range(4):
        fire_gather(k, k)

    def round_(w, carry):
        h = (w % 2) * WB
        nh = ((w + 1) % 2) * WB

        for k in range(8):
            drain_gather(k)
            fire_scatter(h + k, k)
            s2 = (k + 4) % 8
            if k == 0:
                @pl.when(w < GW - 1)
                def _():
                    fetch_idx_g(w + 1)
            if k < 4:
                @pl.when(w > 0)
                def _():
                    wait_scatter(s2)
                fire_gather(h + k + 4, s2)
            else:
                wait_scatter(s2)
                if k == 4:
                    @pl.when(w < GW - 1)
                    def _():
                        fetch_idx_d(w + 1)
                        wait_idx_g()
                @pl.when(w < GW - 1)
                def _():
                    fire_gather(nh + k - 4, s2)
                if k == 7:
                    @pl.when(w < GW - 1)
                    def _():
                        wait_idx_d()
        return carry

    lax.fori_loop(0, GW, round_, 0)
    for k in range(4, 8):
        wait_scatter(k)
    plsc.subcore_barrier()

    @pl.when(s < NS - 1)
    def _():
        pltpu.sync_copy(acc.at[pl.ds(s * RPT_A, RPT_A)],
                        out_hbm.at[pl.ds(c * NN + s * RPT_A, RPT_A)])

    @pl.when(s == NS - 1)
    def _():
        pltpu.sync_copy(acc.at[pl.ds((NS - 1) * RPT_A, RPT_L)],
                        out_hbm.at[pl.ds(c * NN + (NS - 1) * RPT_A, RPT_L)])


@functools.cache
def _make_sc_layer():
    return pl.kernel(
        _sc_layer_body,
        out_type=jax.ShapeDtypeStruct((NC * NN, DD), jnp.float32),
        mesh=plsc.VectorSubcoreMesh(core_axis_name="c", subcore_axis_name="s",
                                    num_cores=NC, num_subcores=NS),
        scratch_types=[
            pltpu.VMEM((2 * WB, BB), jnp.int32),
            pltpu.VMEM((2 * WB, BB), jnp.int32),
            pltpu.VMEM((KK, BB, DD), jnp.float32),
            pltpu.VMEM_SHARED((NN + 8, DD), jnp.float32),
        ] + [pltpu.SemaphoreType.DMA] * 18,
    )


def _sc_layer(*args):
    return _make_sc_layer()(*args)


def kernel(features0, features1, edge_index, edge_type, rel_x, rel_edge_index,
           rel_edge_attr, W1_0, b1_0, W1_1, b1_1, Wn, bn, W2, b2):
    s_idx = edge_index[0].astype(jnp.int32)
    d_idx = edge_index[1].astype(jnp.int32)
    t_idx = edge_type.astype(jnp.int32)
    pad = JBLK * BB - EPW
    pad_g = (jnp.arange(NW * pad, dtype=jnp.int32) * 331) % (RR * NN)
    pad_d = PAD_ROW + (jnp.arange(NW * pad, dtype=jnp.int32) % 8)
    gidx = jnp.concatenate(
        [(t_idx * NN + s_idx).reshape(NW, EPW),
         pad_g.reshape(NW, pad)], axis=1).reshape(NW, JBLK, BB)
    didx = jnp.concatenate(
        [d_idx.reshape(NW, EPW),
         pad_d.reshape(NW, pad)], axis=1).reshape(NW, JBLK, BB)
    zeros = jnp.zeros((NN, DD), jnp.float32)

    h0, wr = _prologue(features0, features1, rel_x, rel_edge_index,
                       rel_edge_attr, Wn, bn, W1_0, b1_0, W1_1, b1_1)
    h0 = h0.reshape(RR * NN, DD)
    p1 = _sc_layer(h0, gidx, didx, zeros).reshape(NC, NN, DD)
    h1 = _mid(p1, wr).reshape(RR * NN, DD)
    p2 = _sc_layer(h1, gidx, didx, zeros).reshape(NC, NN, DD)
    return _epilogue(p2, W2, b2)
